# baseline (speedup 1.0000x reference)
import functools

import jax
import jax.numpy as jnp
from jax import lax
from jax.experimental import pallas as pl
from jax.experimental.pallas import tpu as pltpu
from jax.experimental.pallas import tpu_sc as plsc

B = 16384
USER_DIM = 128
LOC_DIM = 64
K = 64
EPS = 1e-08
BM = 2048
GRID = B // BM

NC, NS, NL = 2, 16, 16
NW = NC * NS
BPW = B // NW
RW = 128


def _route_body(fu, w1, b1, w2, b2, w3, b3, thb, idx_ref, bias_ref):
    xt = fu[...].T
    h = jnp.dot(w1[...], xt, preferred_element_type=jnp.float32) + b1[...]
    h = jnp.maximum(h, 0.01 * h)
    h = jnp.dot(w2[...], h, preferred_element_type=jnp.float32) + b2[...]
    h = jnp.maximum(h, 0.01 * h)
    logits = jnp.dot(w3[...], h, preferred_element_type=jnp.float32) + b3[...]
    colmax = jnp.max(logits, axis=0, keepdims=True)
    iota = lax.broadcasted_iota(jnp.int32, (K, BM), 0)
    first = jnp.min(jnp.where(logits == colmax, iota, K), axis=0, keepdims=True)
    idx_ref[...] = first
    onehot = (iota == first).astype(jnp.float32)
    dn = (((1,), (0,)), ((), ()))
    bias_ref[...] = lax.dot_general(thb[...], onehot, dn,
                                    preferred_element_type=jnp.float32,
                                    precision=lax.Precision.HIGHEST)


def _sc_body(th_hbm, idx_hbm, g_hbm, idx_v, rows_v, sem):
    wid = lax.axis_index("s") * NC + lax.axis_index("c")
    base = wid * BPW
    pltpu.sync_copy(idx_hbm.at[pl.ds(base, BPW)], idx_v)
    pltpu.async_copy(th_hbm.at[idx_v], rows_v, sem).wait()
    pltpu.sync_copy(rows_v, g_hbm.at[pl.ds(base, BPW)])


def _tail_body(g, fl, bias, fpt, out_ref, acc_ref):
    i = pl.program_id(0)

    @pl.when(i == 0)
    def _init():
        acc_ref[0] = 0.0

    gt = g[...].T
    locT = fl[...].T
    bb = bias[...]
    mu = jnp.sum(gt[0:LOC_DIM] * locT, axis=0, keepdims=True) + bb[0:1]
    sd = jnp.sum(gt[LOC_DIM:2 * LOC_DIM] * locT, axis=0, keepdims=True) \
        + bb[1:2]
    d = jnp.abs(sd) + EPS
    r = (mu - fpt[...]) / d
    acc_ref[0] += jnp.sum(jnp.log(d) - 0.5 * (r * r))

    @pl.when(i == GRID - 1)
    def _fin():
        out_ref[0, 0] = -acc_ref[0] / B


@jax.jit
def _run(feat_user, feat_loc, feat_price, W1, b1, W2, b2, W3, b3, theta):
    full = lambda a: pl.BlockSpec(a.shape, lambda i: (0,) * a.ndim)
    thb = theta[:, :, LOC_DIM].T
    rargs = (feat_user, W1, b1.reshape(32, 1), W2, b2.reshape(16, 1),
             W3, b3.reshape(K, 1), thb)
    idx2d, bias = pl.pallas_call(
        _route_body,
        grid=(GRID,),
        in_specs=[pl.BlockSpec((BM, USER_DIM), lambda i: (i, 0))] +
                 [full(a) for a in rargs[1:]],
        out_specs=[pl.BlockSpec((1, BM), lambda i: (0, i)),
                   pl.BlockSpec((2, BM), lambda i: (0, i))],
        out_shape=[jax.ShapeDtypeStruct((1, B), jnp.int32),
                   jax.ShapeDtypeStruct((2, B), jnp.float32)],
    )(*rargs)
    idx = idx2d.reshape(B)
    th2 = jnp.concatenate(
        [theta[:, 0, :LOC_DIM], theta[:, 1, :LOC_DIM]], axis=1)

    g = pl.kernel(
        _sc_body,
        mesh=plsc.VectorSubcoreMesh(core_axis_name="c", subcore_axis_name="s"),
        out_type=jax.ShapeDtypeStruct((B, RW), jnp.float32),
        scratch_types=[pltpu.VMEM((BPW,), jnp.int32),
                       pltpu.VMEM((BPW, RW), jnp.float32),
                       pltpu.SemaphoreType.DMA],
    )(th2, idx)

    out = pl.pallas_call(
        _tail_body,
        grid=(GRID,),
        in_specs=[pl.BlockSpec((BM, RW), lambda i: (i, 0)),
                  pl.BlockSpec((BM, LOC_DIM), lambda i: (i, 0)),
                  pl.BlockSpec((2, BM), lambda i: (0, i)),
                  pl.BlockSpec((1, BM), lambda i: (0, i))],
        out_specs=pl.BlockSpec(memory_space=pltpu.SMEM),
        out_shape=jax.ShapeDtypeStruct((1, 1), jnp.float32),
        scratch_shapes=[pltpu.SMEM((1,), jnp.float32)],
    )(g, feat_loc, bias, feat_price.reshape(1, B))
    return out[0, 0]


def kernel(feat_user, feat_loc, feat_price, W1, b1, W2, b2, W3, b3, theta):
    return _run(feat_user, feat_loc, feat_price, W1, b1, W2, b2, W3, b3, theta)


# speedup vs baseline: 11.7674x; 11.7674x over previous
import jax
import jax.numpy as jnp
from jax import lax
from jax.experimental import pallas as pl
from jax.experimental.pallas import tpu as pltpu

B = 16384
USER_DIM = 128
LOC_DIM = 64
K = 64
EPS = 1e-08
BM = 2048
GRID = B // BM


def _body(fu, fl, fpt, w1, b1, w2, b2, w3, b3,
          thmu_w, thmu_b, thsd_w, thsd_b, out_ref, acc_ref):
    i = pl.program_id(0)

    @pl.when(i == 0)
    def _init():
        acc_ref[0] = 0.0

    xt = fu[...].T
    h = jnp.dot(w1[...], xt, preferred_element_type=jnp.float32) + b1[...]
    h = jnp.maximum(h, 0.01 * h)
    h = jnp.dot(w2[...], h, preferred_element_type=jnp.float32) + b2[...]
    h = jnp.maximum(h, 0.01 * h)
    logits = jnp.dot(w3[...], h, preferred_element_type=jnp.float32) + b3[...]

    colmax = jnp.max(logits, axis=0, keepdims=True)
    iota = lax.broadcasted_iota(jnp.int32, (K, BM), 0)
    first = jnp.min(jnp.where(logits == colmax, iota, K), axis=0, keepdims=True)
    onehot = iota == first

    loc = fl[...]
    dn = (((1,), (1,)), ((), ()))
    mu_all = lax.dot_general(thmu_w[...], loc, dn,
                             preferred_element_type=jnp.float32,
                             precision=lax.Precision.HIGHEST) + thmu_b[...]
    sd_all = lax.dot_general(thsd_w[...], loc, dn,
                             preferred_element_type=jnp.float32,
                             precision=lax.Precision.HIGHEST) + thsd_b[...]
    mu = jnp.sum(jnp.where(onehot, mu_all, 0.0), axis=0, keepdims=True)
    sd = jnp.sum(jnp.where(onehot, sd_all, 0.0), axis=0, keepdims=True)

    d = jnp.abs(sd) + EPS
    r = (mu - fpt[...]) / d
    acc_ref[0] += jnp.sum(jnp.log(d) - 0.5 * (r * r))

    @pl.when(i == GRID - 1)
    def _fin():
        out_ref[0, 0] = -acc_ref[0] / B


@jax.jit
def _run(feat_user, feat_loc, feat_price, W1, b1, W2, b2, W3, b3, theta):
    thmu_w = theta[:, 0, :LOC_DIM]
    thsd_w = theta[:, 1, :LOC_DIM]
    thmu_b = theta[:, 0, LOC_DIM].reshape(K, 1)
    thsd_b = theta[:, 1, LOC_DIM].reshape(K, 1)

    full = lambda a: pl.BlockSpec(a.shape, lambda i: (0,) * a.ndim)
    args = (feat_user, feat_loc, feat_price.reshape(1, B),
            W1, b1.reshape(32, 1), W2, b2.reshape(16, 1),
            W3, b3.reshape(K, 1), thmu_w, thmu_b, thsd_w, thsd_b)
    in_specs = [pl.BlockSpec((BM, USER_DIM), lambda i: (i, 0)),
                pl.BlockSpec((BM, LOC_DIM), lambda i: (i, 0)),
                pl.BlockSpec((1, BM), lambda i: (0, i))] + \
        [full(a) for a in args[3:]]
    out = pl.pallas_call(
        _body,
        grid=(GRID,),
        in_specs=in_specs,
        out_specs=pl.BlockSpec(memory_space=pltpu.SMEM),
        out_shape=jax.ShapeDtypeStruct((1, 1), jnp.float32),
        scratch_shapes=[pltpu.SMEM((1,), jnp.float32)],
    )(*args)
    return out[0, 0]


def kernel(feat_user, feat_loc, feat_price, W1, b1, W2, b2, W3, b3, theta):
    return _run(feat_user, feat_loc, feat_price, W1, b1, W2, b2, W3, b3, theta)


# speedup vs baseline: 12.1512x; 1.0326x over previous
import jax
import jax.numpy as jnp
from jax import lax
from jax.experimental import pallas as pl
from jax.experimental.pallas import tpu as pltpu

B = 16384
USER_DIM = 128
LOC_DIM = 64
K = 64
EPS = 1e-08
BM = 4096
GRID = B // BM


def _body(fu, fl, fpt, w1, b1, w2, b2, w3, b3,
          thmu_w, thmu_b, thsd_w, thsd_b, out_ref, acc_ref):
    i = pl.program_id(0)

    @pl.when(i == 0)
    def _init():
        acc_ref[0] = 0.0

    xt = fu[...].T
    h = jnp.dot(w1[...], xt, preferred_element_type=jnp.float32) + b1[...]
    h = jnp.maximum(h, 0.01 * h)
    h = jnp.dot(w2[...], h, preferred_element_type=jnp.float32) + b2[...]
    h = jnp.maximum(h, 0.01 * h)
    logits = jnp.dot(w3[...], h, preferred_element_type=jnp.float32) + b3[...]

    colmax = jnp.max(logits, axis=0, keepdims=True)
    iota = lax.broadcasted_iota(jnp.int32, (K, BM), 0)
    first = jnp.min(jnp.where(logits == colmax, iota, K), axis=0, keepdims=True)
    onehot = iota == first

    loc = fl[...]
    dn = (((1,), (1,)), ((), ()))
    mu_all = lax.dot_general(thmu_w[...], loc, dn,
                             preferred_element_type=jnp.float32,
                             precision=lax.Precision.HIGHEST) + thmu_b[...]
    sd_all = lax.dot_general(thsd_w[...], loc, dn,
                             preferred_element_type=jnp.float32,
                             precision=lax.Precision.HIGHEST) + thsd_b[...]
    mu = jnp.sum(jnp.where(onehot, mu_all, 0.0), axis=0, keepdims=True)
    sd = jnp.sum(jnp.where(onehot, sd_all, 0.0), axis=0, keepdims=True)

    d = jnp.abs(sd) + EPS
    r = (mu - fpt[...]) / d
    acc_ref[0] += jnp.sum(jnp.log(d) - 0.5 * (r * r))

    @pl.when(i == GRID - 1)
    def _fin():
        out_ref[0, 0] = -acc_ref[0] / B


@jax.jit
def _run(feat_user, feat_loc, feat_price, W1, b1, W2, b2, W3, b3, theta):
    thmu_w = theta[:, 0, :LOC_DIM]
    thsd_w = theta[:, 1, :LOC_DIM]
    thmu_b = theta[:, 0, LOC_DIM].reshape(K, 1)
    thsd_b = theta[:, 1, LOC_DIM].reshape(K, 1)

    full = lambda a: pl.BlockSpec(a.shape, lambda i: (0,) * a.ndim)
    args = (feat_user, feat_loc, feat_price.reshape(1, B),
            W1, b1.reshape(32, 1), W2, b2.reshape(16, 1),
            W3, b3.reshape(K, 1), thmu_w, thmu_b, thsd_w, thsd_b)
    in_specs = [pl.BlockSpec((BM, USER_DIM), lambda i: (i, 0)),
                pl.BlockSpec((BM, LOC_DIM), lambda i: (i, 0)),
                pl.BlockSpec((1, BM), lambda i: (0, i))] + \
        [full(a) for a in args[3:]]
    out = pl.pallas_call(
        _body,
        grid=(GRID,),
        in_specs=in_specs,
        out_specs=pl.BlockSpec(memory_space=pltpu.SMEM),
        out_shape=jax.ShapeDtypeStruct((1, 1), jnp.float32),
        scratch_shapes=[pltpu.SMEM((1,), jnp.float32)],
    )(*args)
    return out[0, 0]


def kernel(feat_user, feat_loc, feat_price, W1, b1, W2, b2, W3, b3, theta):
    return _run(feat_user, feat_loc, feat_price, W1, b1, W2, b2, W3, b3, theta)
